# Initial kernel scaffold; baseline (speedup 1.0000x reference)
import jax
import jax.numpy as jnp
from jax.experimental import pallas as pl


def kernel(x, edge_index, batch, gn0_w, gn0_b, gn0_ms, gn1_w, gn1_b, gn1_ms, gn2_w, gn2_b, gn2_ms, gn3_w, gn3_b, gn3_ms, W1, b1, W2, b2, W3, b3, gW1, gb1, gW2, gb2, gW3, gb3, lW1, lb1, lW2, lb2, lW3, lb3):
    raise NotImplementedError("write your pallas kernel here")


# speedup vs baseline: 6.1392x; 6.1392x over previous
import functools

import jax
import jax.numpy as jnp
from jax.experimental import pallas as pl
from jax.experimental.pallas import tpu as pltpu
from jax.experimental.pallas import tpu_sc as plsc

N = 100000
E = 1600000
G = 64
DIN = 4
H = 128
EPS = 1e-5

NC = 2
NS = 16
NW = NC * NS
EPT = E // NW
EC = 1000
SRCB = 1008
NCH = EPT // EC
R8 = 6256
ZB = 391
NF = H // 16

NBN = 5000
NBLK = N // NBN

_f32 = jnp.float32


@functools.cache
def _mesh():
    return plsc.VectorSubcoreMesh(core_axis_name="c", subcore_axis_name="s",
                                  num_cores=NC, num_subcores=NS)


def _row_lo(s):
    return pl.multiple_of(jnp.minimum(s * R8, N - R8), 8)


def _deg_body(dst_hbm, ones_hbm, zeros_hbm, out_hbm, acc, dstv, onesv):
    c = jax.lax.axis_index("c")
    s = jax.lax.axis_index("s")
    tid = c * NS + s
    ebase = tid * EPT
    rlo = _row_lo(s)
    pltpu.sync_copy(ones_hbm, onesv)
    pltpu.sync_copy(zeros_hbm, acc.at[pl.ds(rlo, R8)])
    plsc.subcore_barrier()

    def body(j, carry):
        base = ebase + j * EC
        pltpu.sync_copy(dst_hbm.at[pl.ds(base, EC)], dstv)
        pltpu.sync_copy(onesv, acc.at[dstv], add=True)
        return carry

    jax.lax.fori_loop(0, NCH, body, 0)
    plsc.subcore_barrier()
    pltpu.sync_copy(acc.at[pl.ds(rlo, R8)], out_hbm.at[c, pl.ds(rlo, R8)])


_SC_PARAMS = pltpu.CompilerParams(use_tc_tiling_on_sc=False)


def _deg_call(dst, ones_ec, zeros_rpt):
    return pl.kernel(
        _deg_body,
        out_type=jax.ShapeDtypeStruct((NC, N, 1), _f32),
        mesh=_mesh(),
        compiler_params=_SC_PARAMS,
        scratch_types=[
            pltpu.VMEM_SHARED((N, 1), _f32),
            pltpu.VMEM((EC,), jnp.int32),
            pltpu.VMEM((EC, 1), _f32),
        ],
    )(dst, ones_ec, zeros_rpt)


def _mp_body(gt_hbm, src_hbm, dst_hbm, out_hbm, acc, srcv, dstv, rows, zbuf):
    c = jax.lax.axis_index("c")
    s = jax.lax.axis_index("s")
    tid = c * NS + s
    ebase = tid * EPT
    rlo = _row_lo(s)

    def zinit(i, carry):
        zbuf[i, :] = jnp.zeros((16,), _f32)
        return carry

    jax.lax.fori_loop(0, ZB, zinit, 0)

    for f in range(NF):
        for z in range(R8 // ZB):
            pltpu.sync_copy(zbuf, acc.at[pl.ds(rlo + z * ZB, ZB)])
        plsc.subcore_barrier()

        def body(j, carry):
            base = ebase + j * EC
            pltpu.sync_copy(src_hbm.at[pl.ds(base, EC)], srcv.at[pl.ds(0, EC)])
            pltpu.sync_copy(dst_hbm.at[pl.ds(base, EC)], dstv)

            def shift(k, c2):
                sl = pl.ds(k * 16, 16)
                srcv[sl] = srcv[sl] + f * N
                return c2

            jax.lax.fori_loop(0, SRCB // 16, shift, 0)
            pltpu.sync_copy(gt_hbm.at[srcv.at[pl.ds(0, EC)]], rows)
            pltpu.sync_copy(rows, acc.at[dstv], add=True)
            return carry

        jax.lax.fori_loop(0, NCH, body, 0)
        plsc.subcore_barrier()
        pltpu.sync_copy(acc.at[pl.ds(rlo, R8)],
                        out_hbm.at[c, f, pl.ds(rlo, R8), :])
        plsc.subcore_barrier()


def _mp_call(g, src, dst):
    gt = g.reshape(N, NF, 16).transpose(1, 0, 2).reshape(NF * N, 16)
    mp4 = pl.kernel(
        _mp_body,
        out_type=jax.ShapeDtypeStruct((NC, NF, N, 16), _f32),
        mesh=_mesh(),
        compiler_params=_SC_PARAMS,
        scratch_types=[
            pltpu.VMEM_SHARED((N, 16), _f32),
            pltpu.VMEM((SRCB,), jnp.int32),
            pltpu.VMEM((EC,), jnp.int32),
            pltpu.VMEM((EC, 16), _f32),
            pltpu.VMEM((ZB, 16), _f32),
        ],
    )(gt, src, dst)
    return mp4.transpose(0, 2, 1, 3).reshape(NC, N, H)


def _oh(bb):
    gi = jax.lax.broadcasted_iota(jnp.int32, (G, NBN), 0)
    return (gi == bb[None, :]).astype(_f32)


def _ohT(bb):
    gi = jax.lax.broadcasted_iota(jnp.int32, (NBN, G), 1)
    return (gi == bb[:, None]).astype(_f32)


def _stats0_body(x_ref, b3_ref, sum_ref, sq_ref, cnt_ref):
    i = pl.program_id(0)
    xb = x_ref[...]
    bb = b3_ref[0, 0, :]
    oh = _oh(bb)

    @pl.when(i == 0)
    def _():
        sum_ref[...] = jnp.zeros_like(sum_ref)
        sq_ref[...] = jnp.zeros_like(sq_ref)
        cnt_ref[...] = jnp.zeros_like(cnt_ref)

    sum_ref[...] += jnp.dot(oh, xb, preferred_element_type=_f32)
    sq_ref[...] += jnp.dot(oh, xb * xb, preferred_element_type=_f32)
    cnt_ref[...] += jnp.sum(oh, axis=1, keepdims=True)


def _norm_tables(ssum, ssq, cnt, w, b, ms):
    c = jnp.clip(cnt, 1.0)
    mean_t = ssum / c
    var_t = ssq / c + (ms * ms - 2.0 * ms) * mean_t * mean_t
    scale_t = w / jnp.sqrt(var_t + EPS)
    shift_t = mean_t * ms * scale_t
    return scale_t, shift_t


def _h1_body(x_ref, b3_ref, degp_ref, xsum_ref, xsq_ref, cnt_ref,
             gw_ref, gb_ref, gms_ref, w1_ref, g_ref, dinv_ref):
    xb = x_ref[...]
    bb = b3_ref[0, 0, :]
    ohT = _ohT(bb)
    scale_t, shift_t = _norm_tables(xsum_ref[...], xsq_ref[...], cnt_ref[...],
                                    gw_ref[...], gb_ref[...], gms_ref[...])
    ns = jnp.dot(ohT, scale_t, preferred_element_type=_f32)
    nsh = jnp.dot(ohT, shift_t, preferred_element_type=_f32)
    xn = xb * ns - nsh + gb_ref[...]
    h = jnp.dot(xn, w1_ref[...], preferred_element_type=_f32)
    deg = degp_ref[0, :, 0] + degp_ref[1, :, 0] + 1.0
    dinv = jax.lax.rsqrt(deg)
    g_ref[...] = h * dinv[:, None]
    dinv_ref[0, 0, :] = dinv


def _statsc_body(mp_ref, g_ref, dinv_ref, b3_ref, bias_ref,
                 o_ref, sum_ref, sq_ref):
    i = pl.program_id(0)
    m = mp_ref[0] + mp_ref[1]
    dinv = dinv_ref[0, 0, :]
    o = (m + g_ref[...]) * dinv[:, None] + bias_ref[...]
    o_ref[...] = o
    bb = b3_ref[0, 0, :]
    oh = _oh(bb)

    @pl.when(i == 0)
    def _():
        sum_ref[...] = jnp.zeros_like(sum_ref)
        sq_ref[...] = jnp.zeros_like(sq_ref)

    sum_ref[...] += jnp.dot(oh, o, preferred_element_type=_f32)
    sq_ref[...] += jnp.dot(oh, o * o, preferred_element_type=_f32)


def _next_body(o_ref, b3_ref, ssum_ref, ssq_ref, cnt_ref,
               gw_ref, gb_ref, gms_ref, w_ref, dinv_ref, g_ref):
    bb = b3_ref[0, 0, :]
    ohT = _ohT(bb)
    scale_t, shift_t = _norm_tables(ssum_ref[...], ssq_ref[...], cnt_ref[...],
                                    gw_ref[...], gb_ref[...], gms_ref[...])
    ns = jnp.dot(ohT, scale_t, preferred_element_type=_f32)
    nsh = jnp.dot(ohT, shift_t, preferred_element_type=_f32)
    xn = jax.nn.relu(o_ref[...] * ns - nsh + gb_ref[...])
    dinv = dinv_ref[0, 0, :]
    g_ref[...] = jnp.dot(xn, w_ref[...], preferred_element_type=_f32) * dinv[:, None]


def _att_body(o_ref, b3_ref, ssum_ref, ssq_ref, cnt_ref,
              gw_ref, gb_ref, gms_ref,
              gw1_ref, gb1_ref, gw2_ref, gb2_ref, gw3_ref, gb3_ref,
              h_ref, s_ref, ssums_ref):
    i = pl.program_id(0)
    bb = b3_ref[0, 0, :]
    ohT = _ohT(bb)
    scale_t, shift_t = _norm_tables(ssum_ref[...], ssq_ref[...], cnt_ref[...],
                                    gw_ref[...], gb_ref[...], gms_ref[...])
    ns = jnp.dot(ohT, scale_t, preferred_element_type=_f32)
    nsh = jnp.dot(ohT, shift_t, preferred_element_type=_f32)
    h = o_ref[...] * ns - nsh + gb_ref[...]
    h_ref[...] = h
    t = jax.nn.relu(jnp.dot(h, gw1_ref[...], preferred_element_type=_f32) + gb1_ref[...])
    t = jax.nn.relu(jnp.dot(t, gw2_ref[...], preferred_element_type=_f32) + gb2_ref[...])
    sv = jnp.dot(t, gw3_ref[...], preferred_element_type=_f32) + gb3_ref[...]
    s_ref[...] = sv

    @pl.when(i == 0)
    def _():
        ssums_ref[...] = jnp.zeros_like(ssums_ref)

    ssums_ref[...] += jnp.dot(_oh(bb), sv, preferred_element_type=_f32)


def _att2_body(h_ref, s_ref, b3_ref, ssums_ref, cnt_ref, num_ref, den_ref):
    i = pl.program_id(0)
    bb = b3_ref[0, 0, :]
    oh = _oh(bb)
    ohT = _ohT(bb)
    smean = ssums_ref[...] / jnp.clip(cnt_ref[...], 1.0)
    sm_n = jnp.dot(ohT, smean, preferred_element_type=_f32)
    e = jnp.exp(s_ref[...] - sm_n)

    @pl.when(i == 0)
    def _():
        num_ref[...] = jnp.zeros_like(num_ref)
        den_ref[...] = jnp.zeros_like(den_ref)

    num_ref[...] += jnp.dot(oh, e * h_ref[...], preferred_element_type=_f32)
    den_ref[...] += jnp.dot(oh, e, preferred_element_type=_f32)


def _final_body(num_ref, den_ref, lw1_ref, lb1_ref, lw2_ref, lb2_ref,
                lw3_ref, lb3_ref, out_ref):
    p = num_ref[...] / (den_ref[...] + 1e-16)
    z = jax.nn.relu(jnp.dot(p, lw1_ref[...], preferred_element_type=_f32) + lb1_ref[...])
    z = jax.nn.relu(jnp.dot(z, lw2_ref[...], preferred_element_type=_f32) + lb2_ref[...])
    out_ref[...] = jnp.dot(z, lw3_ref[...], preferred_element_type=_f32) + lb3_ref[...]


def _full(shape):
    nd = len(shape)
    return pl.BlockSpec(shape, lambda *_, _nd=nd: (0,) * _nd)


def _nblock(width):
    return pl.BlockSpec((NBN, width), lambda i: (i, 0))


_B3 = pl.BlockSpec((1, 1, NBN), lambda i: (i, 0, 0))


def kernel(x, edge_index, batch, gn0_w, gn0_b, gn0_ms, gn1_w, gn1_b, gn1_ms,
           gn2_w, gn2_b, gn2_ms, gn3_w, gn3_b, gn3_ms, W1, b1, W2, b2, W3, b3,
           gW1, gb1, gW2, gb2, gW3, gb3, lW1, lb1, lW2, lb2, lW3, lb3):
    src = edge_index[0]
    dst = edge_index[1]
    batch3 = batch.reshape(NBLK, 1, NBN)
    ones_ec = jnp.ones((EC, 1), _f32)
    zeros_r8 = jnp.zeros((R8, 1), _f32)

    degp = _deg_call(dst, ones_ec, zeros_r8)

    xsum, xsq, cnt = pl.pallas_call(
        _stats0_body,
        grid=(NBLK,),
        in_specs=[_nblock(DIN), _B3],
        out_specs=[_full((G, DIN)), _full((G, DIN)), _full((G, 1))],
        out_shape=[jax.ShapeDtypeStruct((G, DIN), _f32),
                   jax.ShapeDtypeStruct((G, DIN), _f32),
                   jax.ShapeDtypeStruct((G, 1), _f32)],
    )(x, batch3)

    g1, dinv3 = pl.pallas_call(
        _h1_body,
        grid=(NBLK,),
        in_specs=[_nblock(DIN), _B3,
                  pl.BlockSpec((NC, NBN, 1), lambda i: (0, i, 0)),
                  _full((G, DIN)), _full((G, DIN)), _full((G, 1)),
                  _full((1, DIN)), _full((1, DIN)), _full((1, DIN)),
                  _full((DIN, H))],
        out_specs=[_nblock(H), _B3],
        out_shape=[jax.ShapeDtypeStruct((N, H), _f32),
                   jax.ShapeDtypeStruct((NBLK, 1, NBN), _f32)],
    )(x, batch3, degp, xsum, xsq, cnt,
      gn0_w.reshape(1, DIN), gn0_b.reshape(1, DIN), gn0_ms.reshape(1, DIN), W1)

    def conv_stats(g, bias, gnp):
        mp = _mp_call(g, src, dst)
        return pl.pallas_call(
            _statsc_body,
            grid=(NBLK,),
            in_specs=[pl.BlockSpec((NC, NBN, H), lambda i: (0, i, 0)),
                      _nblock(H), _B3, _B3, _full((1, H))],
            out_specs=[_nblock(H), _full((G, H)), _full((G, H))],
            out_shape=[jax.ShapeDtypeStruct((N, H), _f32),
                       jax.ShapeDtypeStruct((G, H), _f32),
                       jax.ShapeDtypeStruct((G, H), _f32)],
        )(mp, g, dinv3, batch3, bias.reshape(1, H))

    def next_g(o, ssum, ssq, gnw, gnb, gnms, W):
        return pl.pallas_call(
            _next_body,
            grid=(NBLK,),
            in_specs=[_nblock(H), _B3, _full((G, H)), _full((G, H)),
                      _full((G, 1)), _full((1, H)), _full((1, H)),
                      _full((1, H)), _full((H, H)), _B3],
            out_specs=_nblock(H),
            out_shape=jax.ShapeDtypeStruct((N, H), _f32),
        )(o, batch3, ssum, ssq, cnt, gnw.reshape(1, H), gnb.reshape(1, H),
          gnms.reshape(1, H), W, dinv3)

    o1, s1, q1 = conv_stats(g1, b1, None)
    g2 = next_g(o1, s1, q1, gn1_w, gn1_b, gn1_ms, W2)
    o2, s2, q2 = conv_stats(g2, b2, None)
    g3 = next_g(o2, s2, q2, gn2_w, gn2_b, gn2_ms, W3)
    o3, s3, q3 = conv_stats(g3, b3, None)

    h, sc, ssums = pl.pallas_call(
        _att_body,
        grid=(NBLK,),
        in_specs=[_nblock(H), _B3, _full((G, H)), _full((G, H)), _full((G, 1)),
                  _full((1, H)), _full((1, H)), _full((1, H)),
                  _full((H, H)), _full((1, H)),
                  _full((H, H)), _full((1, H)),
                  _full((H, 1)), _full((1, 1))],
        out_specs=[_nblock(H), _nblock(1), _full((G, 1))],
        out_shape=[jax.ShapeDtypeStruct((N, H), _f32),
                   jax.ShapeDtypeStruct((N, 1), _f32),
                   jax.ShapeDtypeStruct((G, 1), _f32)],
    )(o3, batch3, s3, q3, cnt,
      gn3_w.reshape(1, H), gn3_b.reshape(1, H), gn3_ms.reshape(1, H),
      gW1, gb1.reshape(1, H), gW2, gb2.reshape(1, H), gW3, gb3.reshape(1, 1))

    num, den = pl.pallas_call(
        _att2_body,
        grid=(NBLK,),
        in_specs=[_nblock(H), _nblock(1), _B3, _full((G, 1)), _full((G, 1))],
        out_specs=[_full((G, H)), _full((G, 1))],
        out_shape=[jax.ShapeDtypeStruct((G, H), _f32),
                   jax.ShapeDtypeStruct((G, 1), _f32)],
    )(h, sc, batch3, ssums, cnt)

    return pl.pallas_call(
        _final_body,
        in_specs=[_full((G, H)), _full((G, 1)), _full((H, H)), _full((1, H)),
                  _full((H, H)), _full((1, H)), _full((H, 1)), _full((1, 1))],
        out_specs=_full((G, 1)),
        out_shape=jax.ShapeDtypeStruct((G, 1), _f32),
    )(num, den, lW1, lb1.reshape(1, H), lW2, lb2.reshape(1, H),
      lW3, lb3.reshape(1, 1))
